# Initial kernel scaffold; baseline (speedup 1.0000x reference)
import jax
import jax.numpy as jnp
from jax.experimental import pallas as pl


def kernel(x, edge_index, batch, lin0_w, lin0_b, p, msn_scale, mlp_w, mlp_b, bn_g, bn_b, fc1_w, fc1_b, bn4_g, bn4_b, fc2_w, fc2_b):
    raise NotImplementedError("write your pallas kernel here")


# speedup vs baseline: 3.1622x; 3.1622x over previous
import functools

import jax
import jax.numpy as jnp
from jax import lax
from jax.experimental import pallas as pl
from jax.experimental.pallas import tpu as pltpu
from jax.experimental.pallas import tpu_sc as plsc

N_NODES = 10000
N_EDGES = 320000
D_IN = 128
D = 512
N_LAYERS = 3
N_GRAPHS = 64
N_CLASSES = 10
EPS = 1e-7

RB = 1000
NRB = N_NODES // RB
NCH = 4
CW = D // NCH
EB = 80
N_PAD = 10240
ROWS_PER_TILE = N_PAD // 16


def _msg(h, p):
    m = jnp.clip(jax.nn.relu(h) + EPS, EPS, 1e4)
    return jnp.exp(p * jnp.log(m))


def _scal(*vals):
    rows = [jnp.full((CW,), v, jnp.float32) for v in vals]
    rows += [jnp.zeros((CW,), jnp.float32)] * (8 - len(rows))
    return jnp.stack(rows)


def _k0_body(x_ref, w_ref, b_ref, sc_ref, h_ref, m4_ref):
    h = lax.dot_general(x_ref[...], w_ref[...], (((1,), (0,)), ((), ())),
                        preferred_element_type=jnp.float32) + b_ref[...]
    h_ref[...] = h
    m = _msg(h, sc_ref[0, 0])
    for c in range(NCH):
        m4_ref[c] = m[:, c * CW:(c + 1) * CW]


def _lin0_msg(x, w, b2, sc):
    return pl.pallas_call(
        _k0_body,
        grid=(NRB,),
        in_specs=[
            pl.BlockSpec((RB, D_IN), lambda i: (i, 0)),
            pl.BlockSpec((D_IN, D), lambda i: (0, 0)),
            pl.BlockSpec((1, D), lambda i: (0, 0)),
            pl.BlockSpec((8, CW), lambda i: (0, 0)),
        ],
        out_specs=[
            pl.BlockSpec((RB, D), lambda i: (i, 0)),
            pl.BlockSpec((NCH, RB, CW), lambda i: (0, i, 0)),
        ],
        out_shape=[
            jax.ShapeDtypeStruct((N_NODES, D), jnp.float32),
            jax.ShapeDtypeStruct((NCH, N_NODES, CW), jnp.float32),
        ],
    )(x, w, b2, sc)


def _ka_body(h_ref, s4_ref, cnt_ref, w4_ref, b_ref, sc_ref, z_ref, st_ref):
    step = pl.program_id(0)
    inv_p = sc_ref[1, 0]
    scale = sc_ref[2, 0]
    cnt = jnp.maximum(cnt_ref[0][:, :1] + cnt_ref[1][:, :1], 1.0)
    h = h_ref[...]
    xn = jnp.sqrt(jnp.sum(h * h, axis=1, keepdims=True))
    aggs = []
    ssq = jnp.zeros((RB, 1), jnp.float32)
    for c in range(NCH):
        a = jnp.clip(s4_ref[c] / cnt, EPS, 1e4)
        a = jnp.exp(inv_p * jnp.log(a))
        aggs.append(a)
        ssq = ssq + jnp.sum(a * a, axis=1, keepdims=True)
    denom = jnp.maximum(jnp.sqrt(ssq), 1e-12)
    coef = xn * scale / denom
    z = jnp.zeros((RB, D), jnp.float32)
    for c in range(NCH):
        out_c = h[:, c * CW:(c + 1) * CW] + aggs[c] * coef
        z = z + lax.dot_general(out_c, w4_ref[c], (((1,), (0,)), ((), ())),
                                preferred_element_type=jnp.float32)
    z = z + b_ref[...]
    z_ref[...] = z

    @pl.when(step == 0)
    def _():
        st_ref[...] = jnp.zeros_like(st_ref)

    st_ref[0:1, :] += jnp.sum(z, axis=0, keepdims=True)
    st_ref[1:2, :] += jnp.sum(z * z, axis=0, keepdims=True)


def _conv_mlp(h, s4, cnt2, w4, b2, sc):
    return pl.pallas_call(
        _ka_body,
        grid=(NRB,),
        in_specs=[
            pl.BlockSpec((RB, D), lambda i: (i, 0)),
            pl.BlockSpec((NCH, RB, CW), lambda i: (0, i, 0)),
            pl.BlockSpec((2, RB, 8), lambda i: (0, i, 0)),
            pl.BlockSpec((NCH, CW, D), lambda i: (0, 0, 0)),
            pl.BlockSpec((1, D), lambda i: (0, 0)),
            pl.BlockSpec((8, CW), lambda i: (0, 0)),
        ],
        out_specs=[
            pl.BlockSpec((RB, D), lambda i: (i, 0)),
            pl.BlockSpec((8, D), lambda i: (0, 0)),
        ],
        out_shape=[
            jax.ShapeDtypeStruct((N_NODES, D), jnp.float32),
            jax.ShapeDtypeStruct((8, D), jnp.float32),
        ],
    )(h, s4, cnt2, w4, b2, sc)


def _kb_body(has_jk, want_m, want_h, *refs):
    if has_jk:
        z_ref, st_ref, g_ref, b_ref, sc_ref, jk_ref = refs[:6]
        outs = refs[6:]
    else:
        z_ref, st_ref, g_ref, b_ref, sc_ref = refs[:5]
        outs = refs[5:]
    jko_ref = outs[0]
    outs = outs[1:]
    mean = st_ref[0:1, :] / N_NODES
    var = st_ref[1:2, :] / N_NODES - mean * mean
    h = (z_ref[...] - mean) * lax.rsqrt(var + 1e-5) * g_ref[...] + b_ref[...]
    h = jax.nn.relu(h)
    if has_jk:
        jko_ref[...] = jnp.maximum(jk_ref[...], h)
    else:
        jko_ref[...] = h
    if want_h:
        outs[0][...] = h
    if want_m:
        m = _msg(h, sc_ref[0, 0])
        for c in range(NCH):
            outs[1][c] = m[:, c * CW:(c + 1) * CW]


def _bn_relu(z, st, g2, b2, sc, jk, want_m):
    has_jk = jk is not None
    in_specs = [
        pl.BlockSpec((RB, D), lambda i: (i, 0)),
        pl.BlockSpec((8, D), lambda i: (0, 0)),
        pl.BlockSpec((1, D), lambda i: (0, 0)),
        pl.BlockSpec((1, D), lambda i: (0, 0)),
        pl.BlockSpec((8, CW), lambda i: (0, 0)),
    ]
    args = [z, st, g2, b2, sc]
    if has_jk:
        in_specs.append(pl.BlockSpec((RB, D), lambda i: (i, 0)))
        args.append(jk)
    out_specs = [pl.BlockSpec((RB, D), lambda i: (i, 0))]
    out_shape = [jax.ShapeDtypeStruct((N_NODES, D), jnp.float32)]
    if want_m:
        out_specs.append(pl.BlockSpec((RB, D), lambda i: (i, 0)))
        out_shape.append(jax.ShapeDtypeStruct((N_NODES, D), jnp.float32))
        out_specs.append(pl.BlockSpec((NCH, RB, CW), lambda i: (0, i, 0)))
        out_shape.append(jax.ShapeDtypeStruct((NCH, N_NODES, CW), jnp.float32))
    body = functools.partial(_kb_body, has_jk, want_m, want_m)
    return pl.pallas_call(
        body, grid=(NRB,), in_specs=in_specs,
        out_specs=out_specs, out_shape=out_shape,
    )(*args)


def _kc_body(jk_ref, bt_ref, w1_ref, b1_ref, g4_ref, bb4_ref, w2_ref, b2_ref,
             out_ref, gmax_s, gsum_s, gcnt_s):
    step = pl.program_id(0)

    @pl.when(step == 0)
    def _():
        gmax_s[...] = jnp.full_like(gmax_s, -3e38)
        gsum_s[...] = jnp.zeros_like(gsum_s)
        gcnt_s[...] = jnp.zeros_like(gcnt_s)

    h = jk_ref[...]
    ids = bt_ref[...]
    gids = lax.broadcasted_iota(jnp.int32, (1, N_GRAPHS), 1)
    onehot = (ids == gids).astype(jnp.float32)
    gsum_s[...] += lax.dot_general(onehot, h, (((0,), (0,)), ((), ())),
                                   preferred_element_type=jnp.float32)
    ones = jnp.ones((RB, 8), jnp.float32)
    gcnt_s[...] += lax.dot_general(onehot, ones, (((0,), (0,)), ((), ())),
                                   preferred_element_type=jnp.float32)
    for g in range(N_GRAPHS):
        masked = jnp.where(ids == g, h, -3e38)
        gmax_s[g:g + 1, :] = jnp.maximum(
            gmax_s[g:g + 1, :], jnp.max(masked, axis=0, keepdims=True))

    @pl.when(step == NRB - 1)
    def _():
        cnt = gcnt_s[:, :1]
        gmax = jnp.where(cnt > 0, gmax_s[...], 0.0)
        gmean = gsum_s[...] / jnp.maximum(cnt, 1.0)
        y = (lax.dot_general(gmax, w1_ref[0], (((1,), (0,)), ((), ())),
                             preferred_element_type=jnp.float32)
             + lax.dot_general(gmean, w1_ref[1], (((1,), (0,)), ((), ())),
                               preferred_element_type=jnp.float32)
             + b1_ref[...])
        mean = jnp.mean(y, axis=0, keepdims=True)
        var = jnp.mean(y * y, axis=0, keepdims=True) - mean * mean
        y = (y - mean) * lax.rsqrt(var + 1e-5) * g4_ref[...] + bb4_ref[...]
        y = jax.nn.relu(y)
        out_ref[...] = lax.dot_general(y, w2_ref[...], (((1,), (0,)), ((), ())),
                                       preferred_element_type=jnp.float32) + b2_ref[...]


def _pool_head(jk, bt2, w1s, b12, g42, bb42, w2, b22):
    return pl.pallas_call(
        _kc_body,
        grid=(NRB,),
        in_specs=[
            pl.BlockSpec((RB, D), lambda i: (i, 0)),
            pl.BlockSpec((RB, 1), lambda i: (i, 0)),
            pl.BlockSpec((2, D, D), lambda i: (0, 0, 0)),
            pl.BlockSpec((1, D), lambda i: (0, 0)),
            pl.BlockSpec((1, D), lambda i: (0, 0)),
            pl.BlockSpec((1, D), lambda i: (0, 0)),
            pl.BlockSpec((D, N_CLASSES), lambda i: (0, 0)),
            pl.BlockSpec((1, N_CLASSES), lambda i: (0, 0)),
        ],
        out_specs=pl.BlockSpec((N_GRAPHS, N_CLASSES), lambda i: (0, 0)),
        out_shape=jax.ShapeDtypeStruct((N_GRAPHS, N_CLASSES), jnp.float32),
        scratch_shapes=[
            pltpu.VMEM((N_GRAPHS, D), jnp.float32),
            pltpu.VMEM((N_GRAPHS, D), jnp.float32),
            pltpu.VMEM((N_GRAPHS, 8), jnp.float32),
        ],
    )(jk, bt2, w1s, b12, g42, bb42, w2, b22)


@functools.lru_cache(maxsize=None)
def _sc_mesh():
    return plsc.VectorSubcoreMesh(core_axis_name="c", subcore_axis_name="s")

_EDGES_PER_TILE_CNT = N_EDGES // 32
_EDGES_PER_TILE_SEG = N_EDGES // 16


def _cnt_kernel_body(dst_hbm, ones_hbm, zer8_hbm, cnt_hbm, dstv, ones_v, acc):
    c = lax.axis_index("c")
    s = lax.axis_index("s")
    pltpu.sync_copy(ones_hbm, ones_v)
    pltpu.sync_copy(zer8_hbm, acc.at[pl.ds(s * ROWS_PER_TILE, ROWS_PER_TILE)])
    plsc.subcore_barrier()
    base0 = (c * 16 + s) * _EDGES_PER_TILE_CNT

    @pl.loop(0, _EDGES_PER_TILE_CNT // EB)
    def _(k):
        pltpu.sync_copy(dst_hbm.at[pl.ds(base0 + k * EB, EB)], dstv)
        pltpu.sync_copy(ones_v, acc.at[dstv], add=True)

    plsc.subcore_barrier()
    pltpu.sync_copy(acc.at[pl.ds(s * ROWS_PER_TILE, ROWS_PER_TILE)],
                    cnt_hbm.at[c, pl.ds(s * ROWS_PER_TILE, ROWS_PER_TILE)])


def _degree_counts(dst, ones_e, zer8):
    k = pl.kernel(
        _cnt_kernel_body,
        out_type=jax.ShapeDtypeStruct((2, N_PAD, 8), jnp.float32),
        mesh=_sc_mesh(),
        scratch_types=[
            pltpu.VMEM((EB,), jnp.int32),
            pltpu.VMEM((EB, 8), jnp.float32),
            pltpu.VMEM_SHARED((N_PAD, 8), jnp.float32),
        ],
    )
    return k(dst, ones_e, zer8)


def _seg_kernel_body(m4_hbm, src_hbm, dst_hbm, zer_hbm, s4_hbm,
                     srcv, dstv, rows, acc, sem):
    c = lax.axis_index("c")
    s = lax.axis_index("s")

    def do_chunk(chunk):
        pltpu.sync_copy(zer_hbm, acc.at[pl.ds(s * ROWS_PER_TILE, ROWS_PER_TILE)])
        plsc.subcore_barrier()
        base0 = s * _EDGES_PER_TILE_SEG

        @pl.loop(0, _EDGES_PER_TILE_SEG // EB)
        def _(k):
            b = base0 + k * EB
            pltpu.sync_copy(src_hbm.at[pl.ds(b, EB)], srcv)
            pltpu.sync_copy(dst_hbm.at[pl.ds(b, EB)], dstv)
            pltpu.async_copy(m4_hbm.at[chunk].at[srcv], rows, sem).wait()
            pltpu.sync_copy(rows, acc.at[dstv], add=True)

        plsc.subcore_barrier()
        pltpu.sync_copy(acc.at[pl.ds(s * ROWS_PER_TILE, ROWS_PER_TILE)],
                        s4_hbm.at[chunk, pl.ds(s * ROWS_PER_TILE, ROWS_PER_TILE)])
        plsc.subcore_barrier()

    for chunk in range(NCH):
        @pl.when(c == chunk // 2)
        def _(chunk=chunk):
            do_chunk(chunk)


def _segment_sum(m4, src, dst, zer):
    k = pl.kernel(
        _seg_kernel_body,
        out_type=jax.ShapeDtypeStruct((NCH, N_PAD, CW), jnp.float32),
        mesh=_sc_mesh(),
        scratch_types=[
            pltpu.VMEM((EB,), jnp.int32),
            pltpu.VMEM((EB,), jnp.int32),
            pltpu.VMEM((EB, CW), jnp.float32),
            pltpu.VMEM_SHARED((N_PAD, CW), jnp.float32),
            pltpu.SemaphoreType.DMA,
        ],
    )
    return k(m4, src, dst, zer)


def kernel(x, edge_index, batch, lin0_w, lin0_b, p, msn_scale, mlp_w, mlp_b,
           bn_g, bn_b, fc1_w, fc1_b, bn4_g, bn4_b, fc2_w, fc2_b):
    src = edge_index[0]
    dst = edge_index[1]
    ones_e = jnp.ones((EB, 8), jnp.float32)
    zer8 = jnp.zeros((ROWS_PER_TILE, 8), jnp.float32)
    zer = jnp.zeros((ROWS_PER_TILE, CW), jnp.float32)

    cnt2 = _degree_counts(dst, ones_e, zer8)

    h, m4 = _lin0_msg(x, lin0_w, lin0_b[None], _scal(p[0]))
    jk = None
    for i in range(N_LAYERS):
        s4 = _segment_sum(m4, src, dst, zer)
        sc_i = _scal(p[i], 1.0 / p[i], msn_scale[i])
        z, st = _conv_mlp(h, s4, cnt2, mlp_w[i].reshape(NCH, CW, D),
                          mlp_b[i][None], sc_i)
        want_m = i < N_LAYERS - 1
        sc_n = _scal(p[i + 1] if want_m else 0.0)
        outs = _bn_relu(z, st, bn_g[i][None], bn_b[i][None], sc_n, jk, want_m)
        if want_m:
            jk, h, m4 = outs
        else:
            jk = outs[0]

    w1s = fc1_w.reshape(2, D, D)
    out = _pool_head(jk, batch[:, None], w1s, fc1_b[None], bn4_g[None],
                     bn4_b[None], fc2_w, fc2_b[None])
    return out


# speedup vs baseline: 5.7817x; 1.8284x over previous
import functools

import jax
import jax.numpy as jnp
from jax import lax
from jax.experimental import pallas as pl
from jax.experimental.pallas import tpu as pltpu
from jax.experimental.pallas import tpu_sc as plsc

N_NODES = 10000
N_EDGES = 320000
D_IN = 128
D = 512
N_LAYERS = 3
N_GRAPHS = 64
N_CLASSES = 10
EPS = 1e-7

RB = 1000
NRB = N_NODES // RB
NCH = 4
CW = D // NCH
EB = 80
N_PAD = 10240
ROWS_PER_TILE = N_PAD // 16


def _msg(h, p):
    m = jnp.clip(jax.nn.relu(h) + EPS, EPS, 1e4)
    return jnp.exp(p * jnp.log(m))


def _scal(*vals):
    rows = [jnp.full((CW,), v, jnp.float32) for v in vals]
    rows += [jnp.zeros((CW,), jnp.float32)] * (8 - len(rows))
    return jnp.stack(rows)


def _k0_body(x_ref, w_ref, b_ref, sc_ref, h_ref, m4_ref):
    h = lax.dot_general(x_ref[...], w_ref[...], (((1,), (0,)), ((), ())),
                        preferred_element_type=jnp.float32) + b_ref[...]
    h_ref[...] = h
    m = _msg(h, sc_ref[0, 0])
    for c in range(NCH):
        m4_ref[c] = m[:, c * CW:(c + 1) * CW]


def _lin0_msg(x, w, b2, sc):
    return pl.pallas_call(
        _k0_body,
        grid=(NRB,),
        in_specs=[
            pl.BlockSpec((RB, D_IN), lambda i: (i, 0)),
            pl.BlockSpec((D_IN, D), lambda i: (0, 0)),
            pl.BlockSpec((1, D), lambda i: (0, 0)),
            pl.BlockSpec((8, CW), lambda i: (0, 0)),
        ],
        out_specs=[
            pl.BlockSpec((RB, D), lambda i: (i, 0)),
            pl.BlockSpec((NCH, RB, CW), lambda i: (0, i, 0)),
        ],
        out_shape=[
            jax.ShapeDtypeStruct((N_NODES, D), jnp.float32),
            jax.ShapeDtypeStruct((NCH, N_NODES, CW), jnp.float32),
        ],
    )(x, w, b2, sc)


def _ka_body(h_ref, s4_ref, cnt_ref, w4_ref, b_ref, sc_ref, z_ref, st_ref):
    step = pl.program_id(0)
    inv_p = sc_ref[1, 0]
    scale = sc_ref[2, 0]
    cnt = jnp.maximum(cnt_ref[0][:, :1] + cnt_ref[1][:, :1], 1.0)
    h = h_ref[...]
    xn = jnp.sqrt(jnp.sum(h * h, axis=1, keepdims=True))
    aggs = []
    ssq = jnp.zeros((RB, 1), jnp.float32)
    for c in range(NCH):
        a = jnp.clip(s4_ref[c] / cnt, EPS, 1e4)
        a = jnp.exp(inv_p * jnp.log(a))
        aggs.append(a)
        ssq = ssq + jnp.sum(a * a, axis=1, keepdims=True)
    denom = jnp.maximum(jnp.sqrt(ssq), 1e-12)
    coef = xn * scale / denom
    z = jnp.zeros((RB, D), jnp.float32)
    for c in range(NCH):
        out_c = h[:, c * CW:(c + 1) * CW] + aggs[c] * coef
        z = z + lax.dot_general(out_c, w4_ref[c], (((1,), (0,)), ((), ())),
                                preferred_element_type=jnp.float32)
    z = z + b_ref[...]
    z_ref[...] = z

    @pl.when(step == 0)
    def _():
        st_ref[...] = jnp.zeros_like(st_ref)

    st_ref[0:1, :] += jnp.sum(z, axis=0, keepdims=True)
    st_ref[1:2, :] += jnp.sum(z * z, axis=0, keepdims=True)


def _conv_mlp(h, s4, cnt2, w4, b2, sc):
    return pl.pallas_call(
        _ka_body,
        grid=(NRB,),
        in_specs=[
            pl.BlockSpec((RB, D), lambda i: (i, 0)),
            pl.BlockSpec((NCH, RB, CW), lambda i: (0, i, 0)),
            pl.BlockSpec((2, RB, 8), lambda i: (0, i, 0)),
            pl.BlockSpec((NCH, CW, D), lambda i: (0, 0, 0)),
            pl.BlockSpec((1, D), lambda i: (0, 0)),
            pl.BlockSpec((8, CW), lambda i: (0, 0)),
        ],
        out_specs=[
            pl.BlockSpec((RB, D), lambda i: (i, 0)),
            pl.BlockSpec((8, D), lambda i: (0, 0)),
        ],
        out_shape=[
            jax.ShapeDtypeStruct((N_NODES, D), jnp.float32),
            jax.ShapeDtypeStruct((8, D), jnp.float32),
        ],
    )(h, s4, cnt2, w4, b2, sc)


def _kb_body(has_jk, want_m, want_h, *refs):
    if has_jk:
        z_ref, st_ref, g_ref, b_ref, sc_ref, jk_ref = refs[:6]
        outs = refs[6:]
    else:
        z_ref, st_ref, g_ref, b_ref, sc_ref = refs[:5]
        outs = refs[5:]
    jko_ref = outs[0]
    outs = outs[1:]
    mean = st_ref[0:1, :] / N_NODES
    var = st_ref[1:2, :] / N_NODES - mean * mean
    h = (z_ref[...] - mean) * lax.rsqrt(var + 1e-5) * g_ref[...] + b_ref[...]
    h = jax.nn.relu(h)
    if has_jk:
        jko_ref[...] = jnp.maximum(jk_ref[...], h)
    else:
        jko_ref[...] = h
    if want_h:
        outs[0][...] = h
    if want_m:
        m = _msg(h, sc_ref[0, 0])
        for c in range(NCH):
            outs[1][c] = m[:, c * CW:(c + 1) * CW]


def _bn_relu(z, st, g2, b2, sc, jk, want_m):
    has_jk = jk is not None
    in_specs = [
        pl.BlockSpec((RB, D), lambda i: (i, 0)),
        pl.BlockSpec((8, D), lambda i: (0, 0)),
        pl.BlockSpec((1, D), lambda i: (0, 0)),
        pl.BlockSpec((1, D), lambda i: (0, 0)),
        pl.BlockSpec((8, CW), lambda i: (0, 0)),
    ]
    args = [z, st, g2, b2, sc]
    if has_jk:
        in_specs.append(pl.BlockSpec((RB, D), lambda i: (i, 0)))
        args.append(jk)
    out_specs = [pl.BlockSpec((RB, D), lambda i: (i, 0))]
    out_shape = [jax.ShapeDtypeStruct((N_NODES, D), jnp.float32)]
    if want_m:
        out_specs.append(pl.BlockSpec((RB, D), lambda i: (i, 0)))
        out_shape.append(jax.ShapeDtypeStruct((N_NODES, D), jnp.float32))
        out_specs.append(pl.BlockSpec((NCH, RB, CW), lambda i: (0, i, 0)))
        out_shape.append(jax.ShapeDtypeStruct((NCH, N_NODES, CW), jnp.float32))
    body = functools.partial(_kb_body, has_jk, want_m, want_m)
    return pl.pallas_call(
        body, grid=(NRB,), in_specs=in_specs,
        out_specs=out_specs, out_shape=out_shape,
    )(*args)


def _kc_body(jk_ref, bt_ref, w1_ref, b1_ref, g4_ref, bb4_ref, w2_ref, b2_ref,
             out_ref, gmax_s, gsum_s, gcnt_s):
    step = pl.program_id(0)

    @pl.when(step == 0)
    def _():
        gmax_s[...] = jnp.full_like(gmax_s, -3e38)
        gsum_s[...] = jnp.zeros_like(gsum_s)
        gcnt_s[...] = jnp.zeros_like(gcnt_s)

    h = jk_ref[...]
    ids = bt_ref[...]
    gids = lax.broadcasted_iota(jnp.int32, (1, N_GRAPHS), 1)
    onehot = (ids == gids).astype(jnp.float32)
    gsum_s[...] += lax.dot_general(onehot, h, (((0,), (0,)), ((), ())),
                                   preferred_element_type=jnp.float32)
    ones = jnp.ones((RB, 8), jnp.float32)
    gcnt_s[...] += lax.dot_general(onehot, ones, (((0,), (0,)), ((), ())),
                                   preferred_element_type=jnp.float32)
    for g in range(N_GRAPHS):
        masked = jnp.where(ids == g, h, -3e38)
        gmax_s[g:g + 1, :] = jnp.maximum(
            gmax_s[g:g + 1, :], jnp.max(masked, axis=0, keepdims=True))

    @pl.when(step == NRB - 1)
    def _():
        cnt = gcnt_s[:, :1]
        gmax = jnp.where(cnt > 0, gmax_s[...], 0.0)
        gmean = gsum_s[...] / jnp.maximum(cnt, 1.0)
        y = (lax.dot_general(gmax, w1_ref[0], (((1,), (0,)), ((), ())),
                             preferred_element_type=jnp.float32)
             + lax.dot_general(gmean, w1_ref[1], (((1,), (0,)), ((), ())),
                               preferred_element_type=jnp.float32)
             + b1_ref[...])
        mean = jnp.mean(y, axis=0, keepdims=True)
        var = jnp.mean(y * y, axis=0, keepdims=True) - mean * mean
        y = (y - mean) * lax.rsqrt(var + 1e-5) * g4_ref[...] + bb4_ref[...]
        y = jax.nn.relu(y)
        out_ref[...] = lax.dot_general(y, w2_ref[...], (((1,), (0,)), ((), ())),
                                       preferred_element_type=jnp.float32) + b2_ref[...]


def _pool_head(jk, bt2, w1s, b12, g42, bb42, w2, b22):
    return pl.pallas_call(
        _kc_body,
        grid=(NRB,),
        in_specs=[
            pl.BlockSpec((RB, D), lambda i: (i, 0)),
            pl.BlockSpec((RB, 1), lambda i: (i, 0)),
            pl.BlockSpec((2, D, D), lambda i: (0, 0, 0)),
            pl.BlockSpec((1, D), lambda i: (0, 0)),
            pl.BlockSpec((1, D), lambda i: (0, 0)),
            pl.BlockSpec((1, D), lambda i: (0, 0)),
            pl.BlockSpec((D, N_CLASSES), lambda i: (0, 0)),
            pl.BlockSpec((1, N_CLASSES), lambda i: (0, 0)),
        ],
        out_specs=pl.BlockSpec((N_GRAPHS, N_CLASSES), lambda i: (0, 0)),
        out_shape=jax.ShapeDtypeStruct((N_GRAPHS, N_CLASSES), jnp.float32),
        scratch_shapes=[
            pltpu.VMEM((N_GRAPHS, D), jnp.float32),
            pltpu.VMEM((N_GRAPHS, D), jnp.float32),
            pltpu.VMEM((N_GRAPHS, 8), jnp.float32),
        ],
    )(jk, bt2, w1s, b12, g42, bb42, w2, b22)


@functools.lru_cache(maxsize=None)
def _sc_mesh():
    return plsc.VectorSubcoreMesh(core_axis_name="c", subcore_axis_name="s")

_EDGES_PER_TILE_CNT = N_EDGES // 32
_EDGES_PER_TILE_SEG = N_EDGES // 16


def _cnt_kernel_body(dst_hbm, ones_hbm, zer8_hbm, cnt_hbm, dstv, ones_v, acc):
    c = lax.axis_index("c")
    s = lax.axis_index("s")
    pltpu.sync_copy(ones_hbm, ones_v)
    pltpu.sync_copy(zer8_hbm, acc.at[pl.ds(s * ROWS_PER_TILE, ROWS_PER_TILE)])
    plsc.subcore_barrier()
    base0 = (c * 16 + s) * _EDGES_PER_TILE_CNT

    @pl.loop(0, _EDGES_PER_TILE_CNT // EB)
    def _(k):
        pltpu.sync_copy(dst_hbm.at[pl.ds(base0 + k * EB, EB)], dstv)
        pltpu.sync_copy(ones_v, acc.at[dstv], add=True)

    plsc.subcore_barrier()
    pltpu.sync_copy(acc.at[pl.ds(s * ROWS_PER_TILE, ROWS_PER_TILE)],
                    cnt_hbm.at[c, pl.ds(s * ROWS_PER_TILE, ROWS_PER_TILE)])


def _degree_counts(dst, ones_e, zer8):
    k = pl.kernel(
        _cnt_kernel_body,
        out_type=jax.ShapeDtypeStruct((2, N_PAD, 8), jnp.float32),
        mesh=_sc_mesh(),
        scratch_types=[
            pltpu.VMEM((EB,), jnp.int32),
            pltpu.VMEM((EB, 8), jnp.float32),
            pltpu.VMEM_SHARED((N_PAD, 8), jnp.float32),
        ],
    )
    return k(dst, ones_e, zer8)


def _seg_kernel_body(m4_hbm, src_hbm, dst_hbm, zer_hbm, s4_hbm,
                     srcv0, srcv1, dstv0, dstv1, rows0, rows1, acc,
                     is0, is1, gs0, gs1, ss0, ss1):
    c = lax.axis_index("c")
    s = lax.axis_index("s")
    srcv = (srcv0, srcv1)
    dstv = (dstv0, dstv1)
    rows = (rows0, rows1)
    isem = (is0, is1)
    gsem = (gs0, gs1)
    ssem = (ss0, ss1)
    T = _EDGES_PER_TILE_SEG // EB

    def do_chunk(chunk):
        m4c = m4_hbm.at[chunk]
        pltpu.sync_copy(zer_hbm, acc.at[pl.ds(s * ROWS_PER_TILE, ROWS_PER_TILE)])
        plsc.subcore_barrier()
        base0 = s * _EDGES_PER_TILE_SEG

        def idx_start(k, b):
            pltpu.async_copy(src_hbm.at[pl.ds(base0 + k * EB, EB)], srcv[b], isem[b])
            pltpu.async_copy(dst_hbm.at[pl.ds(base0 + k * EB, EB)], dstv[b], isem[b])

        idx_start(0, 0)

        @pl.loop(0, T // 2)
        def _(ko):
            for b in range(2):
                nb = 1 - b
                k = ko * 2 + b
                pltpu.make_async_copy(src_hbm.at[pl.ds(base0, EB)], srcv[b], isem[b]).wait()
                pltpu.make_async_copy(dst_hbm.at[pl.ds(base0, EB)], dstv[b], isem[b]).wait()
                gd = pltpu.async_copy(m4c.at[srcv[b]], rows[b], gsem[b])

                @pl.when(k > 0)
                def _():
                    pltpu.make_async_copy(rows[nb], acc.at[dstv[nb]],
                                          ssem[nb]).wait()

                @pl.when(k + 1 < T)
                def _():
                    idx_start(k + 1, nb)

                gd.wait()
                pltpu.async_copy(rows[b], acc.at[dstv[b]], ssem[b], add=True)

        pltpu.make_async_copy(rows[1], acc.at[dstv[1]], ssem[1]).wait()
        plsc.subcore_barrier()
        pltpu.sync_copy(acc.at[pl.ds(s * ROWS_PER_TILE, ROWS_PER_TILE)],
                        s4_hbm.at[chunk, pl.ds(s * ROWS_PER_TILE, ROWS_PER_TILE)])
        plsc.subcore_barrier()

    for chunk in range(NCH):
        @pl.when(c == chunk // 2)
        def _(chunk=chunk):
            do_chunk(chunk)


def _segment_sum(m4, src, dst, zer):
    k = pl.kernel(
        _seg_kernel_body,
        out_type=jax.ShapeDtypeStruct((NCH, N_PAD, CW), jnp.float32),
        mesh=_sc_mesh(),
        scratch_types=[
            pltpu.VMEM((EB,), jnp.int32),
            pltpu.VMEM((EB,), jnp.int32),
            pltpu.VMEM((EB,), jnp.int32),
            pltpu.VMEM((EB,), jnp.int32),
            pltpu.VMEM((EB, CW), jnp.float32),
            pltpu.VMEM((EB, CW), jnp.float32),
            pltpu.VMEM_SHARED((N_PAD, CW), jnp.float32),
            pltpu.SemaphoreType.DMA,
            pltpu.SemaphoreType.DMA,
            pltpu.SemaphoreType.DMA,
            pltpu.SemaphoreType.DMA,
            pltpu.SemaphoreType.DMA,
            pltpu.SemaphoreType.DMA,
        ],
    )
    return k(m4, src, dst, zer)


def kernel(x, edge_index, batch, lin0_w, lin0_b, p, msn_scale, mlp_w, mlp_b,
           bn_g, bn_b, fc1_w, fc1_b, bn4_g, bn4_b, fc2_w, fc2_b):
    src = edge_index[0]
    dst = edge_index[1]
    ones_e = jnp.ones((EB, 8), jnp.float32)
    zer8 = jnp.zeros((ROWS_PER_TILE, 8), jnp.float32)
    zer = jnp.zeros((ROWS_PER_TILE, CW), jnp.float32)

    cnt2 = _degree_counts(dst, ones_e, zer8)

    h, m4 = _lin0_msg(x, lin0_w, lin0_b[None], _scal(p[0]))
    jk = None
    for i in range(N_LAYERS):
        s4 = _segment_sum(m4, src, dst, zer)
        sc_i = _scal(p[i], 1.0 / p[i], msn_scale[i])
        z, st = _conv_mlp(h, s4, cnt2, mlp_w[i].reshape(NCH, CW, D),
                          mlp_b[i][None], sc_i)
        want_m = i < N_LAYERS - 1
        sc_n = _scal(p[i + 1] if want_m else 0.0)
        outs = _bn_relu(z, st, bn_g[i][None], bn_b[i][None], sc_n, jk, want_m)
        if want_m:
            jk, h, m4 = outs
        else:
            jk = outs[0]

    w1s = fc1_w.reshape(2, D, D)
    out = _pool_head(jk, batch[:, None], w1s, fc1_b[None], bn4_g[None],
                     bn4_b[None], fc2_w, fc2_b[None])
    return out


# speedup vs baseline: 6.0790x; 1.0514x over previous
import functools

import jax
import jax.numpy as jnp
from jax import lax
from jax.experimental import pallas as pl
from jax.experimental.pallas import tpu as pltpu
from jax.experimental.pallas import tpu_sc as plsc

N_NODES = 10000
N_EDGES = 320000
D_IN = 128
D = 512
N_LAYERS = 3
N_GRAPHS = 64
N_CLASSES = 10
EPS = 1e-7

RB = 1000
NRB = N_NODES // RB
NCH = 4
CW = D // NCH
EB = 128
EB_T = 32
EB_C = 80
N_PAD = 10240
ROWS_PER_TILE = N_PAD // 16


def _msg(h, p):
    m = jnp.clip(jax.nn.relu(h) + EPS, EPS, 1e4)
    return jnp.exp(p * jnp.log(m))


def _scal(*vals):
    rows = [jnp.full((CW,), v, jnp.float32) for v in vals]
    rows += [jnp.zeros((CW,), jnp.float32)] * (8 - len(rows))
    return jnp.stack(rows)


def _k0_body(x_ref, w_ref, b_ref, sc_ref, h_ref, m4_ref):
    h = lax.dot_general(x_ref[...], w_ref[...], (((1,), (0,)), ((), ())),
                        preferred_element_type=jnp.float32,
                        precision=lax.Precision.HIGHEST) + b_ref[...]
    h_ref[...] = h
    m = _msg(h, sc_ref[0, 0])
    for c in range(NCH):
        m4_ref[c] = m[:, c * CW:(c + 1) * CW]


def _lin0_msg(x, w, b2, sc):
    return pl.pallas_call(
        _k0_body,
        grid=(NRB,),
        in_specs=[
            pl.BlockSpec((RB, D_IN), lambda i: (i, 0)),
            pl.BlockSpec((D_IN, D), lambda i: (0, 0)),
            pl.BlockSpec((1, D), lambda i: (0, 0)),
            pl.BlockSpec((8, CW), lambda i: (0, 0)),
        ],
        out_specs=[
            pl.BlockSpec((RB, D), lambda i: (i, 0)),
            pl.BlockSpec((NCH, RB, CW), lambda i: (0, i, 0)),
        ],
        out_shape=[
            jax.ShapeDtypeStruct((N_NODES, D), jnp.float32),
            jax.ShapeDtypeStruct((NCH, N_NODES, CW), jnp.float32),
        ],
    )(x, w, b2, sc)


def _ka_body(h_ref, s4_ref, cnt_ref, w4_ref, b_ref, sc_ref, z_ref, st_ref):
    step = pl.program_id(0)
    inv_p = sc_ref[1, 0]
    scale = sc_ref[2, 0]
    cnt = jnp.maximum(cnt_ref[0][:, :1] + cnt_ref[1][:, :1], 1.0)
    h = h_ref[...]
    xn = jnp.sqrt(jnp.sum(h * h, axis=1, keepdims=True))
    aggs = []
    ssq = jnp.zeros((RB, 1), jnp.float32)
    for c in range(NCH):
        a = jnp.clip(s4_ref[c] / cnt, EPS, 1e4)
        a = jnp.exp(inv_p * jnp.log(a))
        aggs.append(a)
        ssq = ssq + jnp.sum(a * a, axis=1, keepdims=True)
    denom = jnp.maximum(jnp.sqrt(ssq), 1e-12)
    coef = xn * scale / denom
    z = jnp.zeros((RB, D), jnp.float32)
    for c in range(NCH):
        out_c = h[:, c * CW:(c + 1) * CW] + aggs[c] * coef
        z = z + lax.dot_general(out_c, w4_ref[c], (((1,), (0,)), ((), ())),
                                preferred_element_type=jnp.float32,
                        precision=lax.Precision.HIGHEST)
    z = z + b_ref[...]
    z_ref[...] = z

    @pl.when(step == 0)
    def _():
        st_ref[...] = jnp.zeros_like(st_ref)

    st_ref[0:1, :] += jnp.sum(z, axis=0, keepdims=True)
    st_ref[1:2, :] += jnp.sum(z * z, axis=0, keepdims=True)


def _conv_mlp(h, s4, cnt2, w4, b2, sc):
    return pl.pallas_call(
        _ka_body,
        grid=(NRB,),
        in_specs=[
            pl.BlockSpec((RB, D), lambda i: (i, 0)),
            pl.BlockSpec((NCH, RB, CW), lambda i: (0, i, 0)),
            pl.BlockSpec((2, RB, CW), lambda i: (0, i, 0)),
            pl.BlockSpec((NCH, CW, D), lambda i: (0, 0, 0)),
            pl.BlockSpec((1, D), lambda i: (0, 0)),
            pl.BlockSpec((8, CW), lambda i: (0, 0)),
        ],
        out_specs=[
            pl.BlockSpec((RB, D), lambda i: (i, 0)),
            pl.BlockSpec((8, D), lambda i: (0, 0)),
        ],
        out_shape=[
            jax.ShapeDtypeStruct((N_NODES, D), jnp.float32),
            jax.ShapeDtypeStruct((8, D), jnp.float32),
        ],
    )(h, s4, cnt2, w4, b2, sc)


def _kb_body(has_jk, want_m, want_h, *refs):
    if has_jk:
        z_ref, st_ref, g_ref, b_ref, sc_ref, jk_ref = refs[:6]
        outs = refs[6:]
    else:
        z_ref, st_ref, g_ref, b_ref, sc_ref = refs[:5]
        outs = refs[5:]
    jko_ref = outs[0]
    outs = outs[1:]
    mean = st_ref[0:1, :] / N_NODES
    var = st_ref[1:2, :] / N_NODES - mean * mean
    h = (z_ref[...] - mean) * lax.rsqrt(var + 1e-5) * g_ref[...] + b_ref[...]
    h = jax.nn.relu(h)
    if has_jk:
        jko_ref[...] = jnp.maximum(jk_ref[...], h)
    else:
        jko_ref[...] = h
    if want_h:
        outs[0][...] = h
    if want_m:
        m = _msg(h, sc_ref[0, 0])
        for c in range(NCH):
            outs[1][c] = m[:, c * CW:(c + 1) * CW]


def _bn_relu(z, st, g2, b2, sc, jk, want_m):
    has_jk = jk is not None
    in_specs = [
        pl.BlockSpec((RB, D), lambda i: (i, 0)),
        pl.BlockSpec((8, D), lambda i: (0, 0)),
        pl.BlockSpec((1, D), lambda i: (0, 0)),
        pl.BlockSpec((1, D), lambda i: (0, 0)),
        pl.BlockSpec((8, CW), lambda i: (0, 0)),
    ]
    args = [z, st, g2, b2, sc]
    if has_jk:
        in_specs.append(pl.BlockSpec((RB, D), lambda i: (i, 0)))
        args.append(jk)
    out_specs = [pl.BlockSpec((RB, D), lambda i: (i, 0))]
    out_shape = [jax.ShapeDtypeStruct((N_NODES, D), jnp.float32)]
    if want_m:
        out_specs.append(pl.BlockSpec((RB, D), lambda i: (i, 0)))
        out_shape.append(jax.ShapeDtypeStruct((N_NODES, D), jnp.float32))
        out_specs.append(pl.BlockSpec((NCH, RB, CW), lambda i: (0, i, 0)))
        out_shape.append(jax.ShapeDtypeStruct((NCH, N_NODES, CW), jnp.float32))
    body = functools.partial(_kb_body, has_jk, want_m, want_m)
    return pl.pallas_call(
        body, grid=(NRB,), in_specs=in_specs,
        out_specs=out_specs, out_shape=out_shape,
    )(*args)


def _kc_body(jk_ref, bt_ref, w1_ref, b1_ref, g4_ref, bb4_ref, w2_ref, b2_ref,
             out_ref, gmax_s, gsum_s, gcnt_s):
    step = pl.program_id(0)

    @pl.when(step == 0)
    def _():
        gmax_s[...] = jnp.full_like(gmax_s, -3e38)
        gsum_s[...] = jnp.zeros_like(gsum_s)
        gcnt_s[...] = jnp.zeros_like(gcnt_s)

    h = jk_ref[...]
    ids = bt_ref[...]
    gids = lax.broadcasted_iota(jnp.int32, (1, N_GRAPHS), 1)
    onehot = (ids == gids).astype(jnp.float32)
    gsum_s[...] += lax.dot_general(onehot, h, (((0,), (0,)), ((), ())),
                                   preferred_element_type=jnp.float32,
                        precision=lax.Precision.HIGHEST)
    ones = jnp.ones((RB, 8), jnp.float32)
    gcnt_s[...] += lax.dot_general(onehot, ones, (((0,), (0,)), ((), ())),
                                   preferred_element_type=jnp.float32,
                        precision=lax.Precision.HIGHEST)
    for g in range(N_GRAPHS):
        masked = jnp.where(ids == g, h, -3e38)
        gmax_s[g:g + 1, :] = jnp.maximum(
            gmax_s[g:g + 1, :], jnp.max(masked, axis=0, keepdims=True))

    @pl.when(step == NRB - 1)
    def _():
        cnt = gcnt_s[:, :1]
        gmax = jnp.where(cnt > 0, gmax_s[...], 0.0)
        gmean = gsum_s[...] / jnp.maximum(cnt, 1.0)
        y = (lax.dot_general(gmax, w1_ref[0], (((1,), (0,)), ((), ())),
                             preferred_element_type=jnp.float32,
                        precision=lax.Precision.HIGHEST)
             + lax.dot_general(gmean, w1_ref[1], (((1,), (0,)), ((), ())),
                               preferred_element_type=jnp.float32,
                        precision=lax.Precision.HIGHEST)
             + b1_ref[...])
        mean = jnp.mean(y, axis=0, keepdims=True)
        var = jnp.mean(y * y, axis=0, keepdims=True) - mean * mean
        y = (y - mean) * lax.rsqrt(var + 1e-5) * g4_ref[...] + bb4_ref[...]
        y = jax.nn.relu(y)
        out_ref[...] = lax.dot_general(y, w2_ref[...], (((1,), (0,)), ((), ())),
                                       preferred_element_type=jnp.float32,
                        precision=lax.Precision.HIGHEST) + b2_ref[...]


def _pool_head(jk, bt2, w1s, b12, g42, bb42, w2, b22):
    return pl.pallas_call(
        _kc_body,
        grid=(NRB,),
        in_specs=[
            pl.BlockSpec((RB, D), lambda i: (i, 0)),
            pl.BlockSpec((RB, 1), lambda i: (i, 0)),
            pl.BlockSpec((2, D, D), lambda i: (0, 0, 0)),
            pl.BlockSpec((1, D), lambda i: (0, 0)),
            pl.BlockSpec((1, D), lambda i: (0, 0)),
            pl.BlockSpec((1, D), lambda i: (0, 0)),
            pl.BlockSpec((D, N_CLASSES), lambda i: (0, 0)),
            pl.BlockSpec((1, N_CLASSES), lambda i: (0, 0)),
        ],
        out_specs=pl.BlockSpec((N_GRAPHS, N_CLASSES), lambda i: (0, 0)),
        out_shape=jax.ShapeDtypeStruct((N_GRAPHS, N_CLASSES), jnp.float32),
        scratch_shapes=[
            pltpu.VMEM((N_GRAPHS, D), jnp.float32),
            pltpu.VMEM((N_GRAPHS, D), jnp.float32),
            pltpu.VMEM((N_GRAPHS, 8), jnp.float32),
        ],
    )(jk, bt2, w1s, b12, g42, bb42, w2, b22)


@functools.lru_cache(maxsize=None)
def _sc_mesh():
    return plsc.VectorSubcoreMesh(core_axis_name="c", subcore_axis_name="s")

_EDGES_PER_TILE_CNT = N_EDGES // 32
_EDGES_PER_TILE_SEG = N_EDGES // 16


def _cnt_kernel_body(dst_hbm, ones_hbm, zer_hbm, cnt_hbm, dstv, ones_v, acc):
    c = lax.axis_index("c")
    s = lax.axis_index("s")
    pltpu.sync_copy(ones_hbm, ones_v)
    pltpu.sync_copy(zer_hbm, acc.at[pl.ds(s * ROWS_PER_TILE, ROWS_PER_TILE)])
    plsc.subcore_barrier()
    base0 = (c * 16 + s) * _EDGES_PER_TILE_CNT

    @pl.loop(0, _EDGES_PER_TILE_CNT // EB_C)
    def _(k):
        pltpu.sync_copy(dst_hbm.at[pl.ds(base0 + k * EB_C, EB_C)], dstv)
        pltpu.sync_copy(ones_v, acc.at[dstv], add=True)

    plsc.subcore_barrier()
    pltpu.sync_copy(acc.at[pl.ds(s * ROWS_PER_TILE, ROWS_PER_TILE)],
                    cnt_hbm.at[c, pl.ds(s * ROWS_PER_TILE, ROWS_PER_TILE)])


def _degree_counts(dst, ones_e, zer8):
    k = pl.kernel(
        _cnt_kernel_body,
        out_type=jax.ShapeDtypeStruct((2, N_PAD, CW), jnp.float32),
        mesh=_sc_mesh(),
        scratch_types=[
            pltpu.VMEM((EB_C,), jnp.int32),
            pltpu.VMEM((EB_C, CW), jnp.float32),
            pltpu.VMEM_SHARED((N_PAD, CW), jnp.float32),
        ],
    )
    return k(dst, ones_e, zer8)


def _seg_kernel_body(m4_hbm, src_hbm, dst_hbm, zer_hbm, s4_hbm,
                     srcv0, srcv1, dstv0, dstv1, rows0, rows1,
                     srcv_t, dstv_t, rows_t, acc,
                     is0, is1, gs0, gs1, ss0, ss1):
    c = lax.axis_index("c")
    s = lax.axis_index("s")
    srcv = (srcv0, srcv1)
    dstv = (dstv0, dstv1)
    rows = (rows0, rows1)
    isem = (is0, is1)
    gsem = (gs0, gs1)
    ssem = (ss0, ss1)
    T = _EDGES_PER_TILE_SEG // EB

    def do_chunk(chunk):
        m4c = m4_hbm.at[chunk]
        pltpu.sync_copy(zer_hbm, acc.at[pl.ds(s * ROWS_PER_TILE, ROWS_PER_TILE)])
        plsc.subcore_barrier()
        base0 = s * _EDGES_PER_TILE_SEG

        def idx_start(k, b):
            pltpu.async_copy(src_hbm.at[pl.ds(base0 + k * EB, EB)], srcv[b], isem[b])
            pltpu.async_copy(dst_hbm.at[pl.ds(base0 + k * EB, EB)], dstv[b], isem[b])

        idx_start(0, 0)

        @pl.loop(0, T // 2)
        def _(ko):
            for b in range(2):
                nb = 1 - b
                k = ko * 2 + b
                pltpu.make_async_copy(src_hbm.at[pl.ds(base0, EB)], srcv[b], isem[b]).wait()
                pltpu.make_async_copy(dst_hbm.at[pl.ds(base0, EB)], dstv[b], isem[b]).wait()
                gd = pltpu.async_copy(m4c.at[srcv[b]], rows[b], gsem[b])

                @pl.when(k > 0)
                def _():
                    pltpu.make_async_copy(rows[nb], acc.at[dstv[nb]],
                                          ssem[nb]).wait()

                @pl.when(k + 1 < T)
                def _():
                    idx_start(k + 1, nb)

                gd.wait()
                pltpu.async_copy(rows[b], acc.at[dstv[b]], ssem[b], add=True)

        pltpu.make_async_copy(rows[1], acc.at[dstv[1]], ssem[1]).wait()
        tb = base0 + T * EB
        pltpu.sync_copy(src_hbm.at[pl.ds(tb, EB_T)], srcv_t)
        pltpu.sync_copy(dst_hbm.at[pl.ds(tb, EB_T)], dstv_t)
        pltpu.async_copy(m4c.at[srcv_t], rows_t, gsem[0]).wait()
        pltpu.sync_copy(rows_t, acc.at[dstv_t], add=True)
        plsc.subcore_barrier()
        pltpu.sync_copy(acc.at[pl.ds(s * ROWS_PER_TILE, ROWS_PER_TILE)],
                        s4_hbm.at[chunk, pl.ds(s * ROWS_PER_TILE, ROWS_PER_TILE)])
        plsc.subcore_barrier()

    for chunk in range(NCH):
        @pl.when(c == chunk // 2)
        def _(chunk=chunk):
            do_chunk(chunk)


def _segment_sum(m4, src, dst, zer):
    k = pl.kernel(
        _seg_kernel_body,
        out_type=jax.ShapeDtypeStruct((NCH, N_PAD, CW), jnp.float32),
        mesh=_sc_mesh(),
        scratch_types=[
            pltpu.VMEM((EB,), jnp.int32),
            pltpu.VMEM((EB,), jnp.int32),
            pltpu.VMEM((EB,), jnp.int32),
            pltpu.VMEM((EB,), jnp.int32),
            pltpu.VMEM((EB, CW), jnp.float32),
            pltpu.VMEM((EB, CW), jnp.float32),
            pltpu.VMEM((EB_T,), jnp.int32),
            pltpu.VMEM((EB_T,), jnp.int32),
            pltpu.VMEM((EB_T, CW), jnp.float32),
            pltpu.VMEM_SHARED((N_PAD, CW), jnp.float32),
            pltpu.SemaphoreType.DMA,
            pltpu.SemaphoreType.DMA,
            pltpu.SemaphoreType.DMA,
            pltpu.SemaphoreType.DMA,
            pltpu.SemaphoreType.DMA,
            pltpu.SemaphoreType.DMA,
        ],
    )
    return k(m4, src, dst, zer)


def kernel(x, edge_index, batch, lin0_w, lin0_b, p, msn_scale, mlp_w, mlp_b,
           bn_g, bn_b, fc1_w, fc1_b, bn4_g, bn4_b, fc2_w, fc2_b):
    src = edge_index[0]
    dst = edge_index[1]
    ones_e = jnp.ones((EB_C, CW), jnp.float32)
    zer = jnp.zeros((ROWS_PER_TILE, CW), jnp.float32)

    cnt2 = _degree_counts(dst, ones_e, zer)

    h, m4 = _lin0_msg(x, lin0_w, lin0_b[None], _scal(p[0]))
    jk = None
    for i in range(N_LAYERS):
        s4 = _segment_sum(m4, src, dst, zer)
        sc_i = _scal(p[i], 1.0 / p[i], msn_scale[i])
        z, st = _conv_mlp(h, s4, cnt2, mlp_w[i].reshape(NCH, CW, D),
                          mlp_b[i][None], sc_i)
        want_m = i < N_LAYERS - 1
        sc_n = _scal(p[i + 1] if want_m else 0.0)
        outs = _bn_relu(z, st, bn_g[i][None], bn_b[i][None], sc_n, jk, want_m)
        if want_m:
            jk, h, m4 = outs
        else:
            jk = outs[0]

    w1s = fc1_w.reshape(2, D, D)
    out = _pool_head(jk, batch[:, None], w1s, fc1_b[None], bn4_g[None],
                     bn4_b[None], fc2_w, fc2_b[None])
    return out


# speedup vs baseline: 6.6182x; 1.0887x over previous
import functools

import jax
import jax.numpy as jnp
from jax import lax
from jax.experimental import pallas as pl
from jax.experimental.pallas import tpu as pltpu
from jax.experimental.pallas import tpu_sc as plsc

N_NODES = 10000
N_EDGES = 320000
D_IN = 128
D = 512
N_LAYERS = 3
N_GRAPHS = 64
N_CLASSES = 10
EPS = 1e-7

RB = 1000
NRB = N_NODES // RB
NCH = 4
CW = D // NCH
EB = 128
EB_T = 32
EB_C = 80
N_PAD = 10240
ROWS_PER_TILE = N_PAD // 16


def _msg(h, p):
    m = jnp.clip(jax.nn.relu(h) + EPS, EPS, 1e4)
    return jnp.exp(p * jnp.log(m))


def _scal(*vals):
    rows = [jnp.full((CW,), v, jnp.float32) for v in vals]
    rows += [jnp.zeros((CW,), jnp.float32)] * (8 - len(rows))
    return jnp.stack(rows)


def _k0_body(x_ref, w_ref, b_ref, sc_ref, h_ref, m4_ref):
    h = lax.dot_general(x_ref[...], w_ref[...], (((1,), (0,)), ((), ())),
                        preferred_element_type=jnp.float32) + b_ref[...]
    h_ref[...] = h
    m = _msg(h, sc_ref[0, 0])
    for c in range(NCH):
        m4_ref[c] = m[:, c * CW:(c + 1) * CW]


def _lin0_msg(x, w, b2, sc):
    return pl.pallas_call(
        _k0_body,
        grid=(NRB,),
        in_specs=[
            pl.BlockSpec((RB, D_IN), lambda i: (i, 0)),
            pl.BlockSpec((D_IN, D), lambda i: (0, 0)),
            pl.BlockSpec((1, D), lambda i: (0, 0)),
            pl.BlockSpec((8, CW), lambda i: (0, 0)),
        ],
        out_specs=[
            pl.BlockSpec((RB, D), lambda i: (i, 0)),
            pl.BlockSpec((NCH, RB, CW), lambda i: (0, i, 0)),
        ],
        out_shape=[
            jax.ShapeDtypeStruct((N_NODES, D), jnp.float32),
            jax.ShapeDtypeStruct((NCH, N_NODES, CW), jnp.float32),
        ],
    )(x, w, b2, sc)


def _ka_body(h_ref, s4_ref, cnt_ref, w4_ref, b_ref, sc_ref, z_ref, st_ref):
    step = pl.program_id(0)
    inv_p = sc_ref[1, 0]
    scale = sc_ref[2, 0]
    cnt = jnp.maximum(cnt_ref[0][:, :1] + cnt_ref[1][:, :1], 1.0)
    h = h_ref[...]
    xn = jnp.sqrt(jnp.sum(h * h, axis=1, keepdims=True))
    aggs = []
    ssq = jnp.zeros((RB, 1), jnp.float32)
    for c in range(NCH):
        a = jnp.clip(s4_ref[c] / cnt, EPS, 1e4)
        a = jnp.exp(inv_p * jnp.log(a))
        aggs.append(a)
        ssq = ssq + jnp.sum(a * a, axis=1, keepdims=True)
    denom = jnp.maximum(jnp.sqrt(ssq), 1e-12)
    coef = xn * scale / denom
    z = jnp.zeros((RB, D), jnp.float32)
    for c in range(NCH):
        out_c = h[:, c * CW:(c + 1) * CW] + aggs[c] * coef
        z = z + lax.dot_general(out_c, w4_ref[c], (((1,), (0,)), ((), ())),
                                preferred_element_type=jnp.float32)
    z = z + b_ref[...]
    z_ref[...] = z

    @pl.when(step == 0)
    def _():
        st_ref[...] = jnp.zeros_like(st_ref)

    st_ref[0:1, :] += jnp.sum(z, axis=0, keepdims=True)
    st_ref[1:2, :] += jnp.sum(z * z, axis=0, keepdims=True)


def _conv_mlp(h, s4, cnt2, w4, b2, sc):
    return pl.pallas_call(
        _ka_body,
        grid=(NRB,),
        in_specs=[
            pl.BlockSpec((RB, D), lambda i: (i, 0)),
            pl.BlockSpec((NCH, RB, CW), lambda i: (0, i, 0)),
            pl.BlockSpec((2, RB, CW), lambda i: (0, i, 0)),
            pl.BlockSpec((NCH, CW, D), lambda i: (0, 0, 0)),
            pl.BlockSpec((1, D), lambda i: (0, 0)),
            pl.BlockSpec((8, CW), lambda i: (0, 0)),
        ],
        out_specs=[
            pl.BlockSpec((RB, D), lambda i: (i, 0)),
            pl.BlockSpec((8, D), lambda i: (0, 0)),
        ],
        out_shape=[
            jax.ShapeDtypeStruct((N_NODES, D), jnp.float32),
            jax.ShapeDtypeStruct((8, D), jnp.float32),
        ],
    )(h, s4, cnt2, w4, b2, sc)


def _kb_body(has_jk, want_m, want_h, *refs):
    if has_jk:
        z_ref, st_ref, g_ref, b_ref, sc_ref, jk_ref = refs[:6]
        outs = refs[6:]
    else:
        z_ref, st_ref, g_ref, b_ref, sc_ref = refs[:5]
        outs = refs[5:]
    jko_ref = outs[0]
    outs = outs[1:]
    mean = st_ref[0:1, :] / N_NODES
    var = st_ref[1:2, :] / N_NODES - mean * mean
    h = (z_ref[...] - mean) * lax.rsqrt(var + 1e-5) * g_ref[...] + b_ref[...]
    h = jax.nn.relu(h)
    if has_jk:
        jko_ref[...] = jnp.maximum(jk_ref[...], h)
    else:
        jko_ref[...] = h
    if want_h:
        outs[0][...] = h
    if want_m:
        m = _msg(h, sc_ref[0, 0])
        for c in range(NCH):
            outs[1][c] = m[:, c * CW:(c + 1) * CW]


def _bn_relu(z, st, g2, b2, sc, jk, want_m):
    has_jk = jk is not None
    in_specs = [
        pl.BlockSpec((RB, D), lambda i: (i, 0)),
        pl.BlockSpec((8, D), lambda i: (0, 0)),
        pl.BlockSpec((1, D), lambda i: (0, 0)),
        pl.BlockSpec((1, D), lambda i: (0, 0)),
        pl.BlockSpec((8, CW), lambda i: (0, 0)),
    ]
    args = [z, st, g2, b2, sc]
    if has_jk:
        in_specs.append(pl.BlockSpec((RB, D), lambda i: (i, 0)))
        args.append(jk)
    out_specs = [pl.BlockSpec((RB, D), lambda i: (i, 0))]
    out_shape = [jax.ShapeDtypeStruct((N_NODES, D), jnp.float32)]
    if want_m:
        out_specs.append(pl.BlockSpec((RB, D), lambda i: (i, 0)))
        out_shape.append(jax.ShapeDtypeStruct((N_NODES, D), jnp.float32))
        out_specs.append(pl.BlockSpec((NCH, RB, CW), lambda i: (0, i, 0)))
        out_shape.append(jax.ShapeDtypeStruct((NCH, N_NODES, CW), jnp.float32))
    body = functools.partial(_kb_body, has_jk, want_m, want_m)
    return pl.pallas_call(
        body, grid=(NRB,), in_specs=in_specs,
        out_specs=out_specs, out_shape=out_shape,
    )(*args)


def _kc_body(jk_ref, bt_ref, w1_ref, b1_ref, g4_ref, bb4_ref, w2_ref, b2_ref,
             out_ref, gmax_s, gsum_s, gcnt_s):
    step = pl.program_id(0)

    @pl.when(step == 0)
    def _():
        gmax_s[...] = jnp.full_like(gmax_s, -3e38)
        gsum_s[...] = jnp.zeros_like(gsum_s)
        gcnt_s[...] = jnp.zeros_like(gcnt_s)

    h = jk_ref[...]
    ids = bt_ref[...]
    gids = lax.broadcasted_iota(jnp.int32, (1, N_GRAPHS), 1)
    onehot = (ids == gids).astype(jnp.float32)
    gsum_s[...] += lax.dot_general(onehot, h, (((0,), (0,)), ((), ())),
                                   preferred_element_type=jnp.float32)
    ones = jnp.ones((RB, 8), jnp.float32)
    gcnt_s[...] += lax.dot_general(onehot, ones, (((0,), (0,)), ((), ())),
                                   preferred_element_type=jnp.float32)
    for g in range(N_GRAPHS):
        masked = jnp.where(ids == g, h, -3e38)
        gmax_s[g:g + 1, :] = jnp.maximum(
            gmax_s[g:g + 1, :], jnp.max(masked, axis=0, keepdims=True))

    @pl.when(step == NRB - 1)
    def _():
        cnt = gcnt_s[:, :1]
        gmax = jnp.where(cnt > 0, gmax_s[...], 0.0)
        gmean = gsum_s[...] / jnp.maximum(cnt, 1.0)
        y = (lax.dot_general(gmax, w1_ref[0], (((1,), (0,)), ((), ())),
                             preferred_element_type=jnp.float32)
             + lax.dot_general(gmean, w1_ref[1], (((1,), (0,)), ((), ())),
                               preferred_element_type=jnp.float32)
             + b1_ref[...])
        mean = jnp.mean(y, axis=0, keepdims=True)
        var = jnp.mean(y * y, axis=0, keepdims=True) - mean * mean
        y = (y - mean) * lax.rsqrt(var + 1e-5) * g4_ref[...] + bb4_ref[...]
        y = jax.nn.relu(y)
        out_ref[...] = lax.dot_general(y, w2_ref[...], (((1,), (0,)), ((), ())),
                                       preferred_element_type=jnp.float32) + b2_ref[...]


def _pool_head(jk, bt2, w1s, b12, g42, bb42, w2, b22):
    return pl.pallas_call(
        _kc_body,
        grid=(NRB,),
        in_specs=[
            pl.BlockSpec((RB, D), lambda i: (i, 0)),
            pl.BlockSpec((RB, 1), lambda i: (i, 0)),
            pl.BlockSpec((2, D, D), lambda i: (0, 0, 0)),
            pl.BlockSpec((1, D), lambda i: (0, 0)),
            pl.BlockSpec((1, D), lambda i: (0, 0)),
            pl.BlockSpec((1, D), lambda i: (0, 0)),
            pl.BlockSpec((D, N_CLASSES), lambda i: (0, 0)),
            pl.BlockSpec((1, N_CLASSES), lambda i: (0, 0)),
        ],
        out_specs=pl.BlockSpec((N_GRAPHS, N_CLASSES), lambda i: (0, 0)),
        out_shape=jax.ShapeDtypeStruct((N_GRAPHS, N_CLASSES), jnp.float32),
        scratch_shapes=[
            pltpu.VMEM((N_GRAPHS, D), jnp.float32),
            pltpu.VMEM((N_GRAPHS, D), jnp.float32),
            pltpu.VMEM((N_GRAPHS, 8), jnp.float32),
        ],
    )(jk, bt2, w1s, b12, g42, bb42, w2, b22)


@functools.lru_cache(maxsize=None)
def _sc_mesh():
    return plsc.VectorSubcoreMesh(core_axis_name="c", subcore_axis_name="s")

_EDGES_PER_TILE_CNT = N_EDGES // 32
_EDGES_PER_TILE_SEG = N_EDGES // 16


def _cnt_kernel_body(dst_hbm, ones_hbm, zer_hbm, cnt_hbm, dstv, ones_v, acc):
    c = lax.axis_index("c")
    s = lax.axis_index("s")
    pltpu.sync_copy(ones_hbm, ones_v)
    pltpu.sync_copy(zer_hbm, acc.at[pl.ds(s * ROWS_PER_TILE, ROWS_PER_TILE)])
    plsc.subcore_barrier()
    base0 = (c * 16 + s) * _EDGES_PER_TILE_CNT

    @pl.loop(0, _EDGES_PER_TILE_CNT // EB_C)
    def _(k):
        pltpu.sync_copy(dst_hbm.at[pl.ds(base0 + k * EB_C, EB_C)], dstv)
        pltpu.sync_copy(ones_v, acc.at[dstv], add=True)

    plsc.subcore_barrier()
    pltpu.sync_copy(acc.at[pl.ds(s * ROWS_PER_TILE, ROWS_PER_TILE)],
                    cnt_hbm.at[c, pl.ds(s * ROWS_PER_TILE, ROWS_PER_TILE)])


def _degree_counts(dst, ones_e, zer8):
    k = pl.kernel(
        _cnt_kernel_body,
        out_type=jax.ShapeDtypeStruct((2, N_PAD, CW), jnp.float32),
        mesh=_sc_mesh(),
        scratch_types=[
            pltpu.VMEM((EB_C,), jnp.int32),
            pltpu.VMEM((EB_C, CW), jnp.float32),
            pltpu.VMEM_SHARED((N_PAD, CW), jnp.float32),
        ],
    )
    return k(dst, ones_e, zer8)


def _seg_kernel_body(m4_hbm, src_hbm, dst_hbm, zer_hbm, s4_hbm,
                     srcv0, srcv1, dstv0, dstv1, rows0, rows1,
                     srcv_t, dstv_t, rows_t, acc,
                     is0, is1, gs0, gs1, ss0, ss1):
    c = lax.axis_index("c")
    s = lax.axis_index("s")
    srcv = (srcv0, srcv1)
    dstv = (dstv0, dstv1)
    rows = (rows0, rows1)
    isem = (is0, is1)
    gsem = (gs0, gs1)
    ssem = (ss0, ss1)
    T = _EDGES_PER_TILE_SEG // EB

    def do_chunk(chunk):
        m4c = m4_hbm.at[chunk]
        pltpu.sync_copy(zer_hbm, acc.at[pl.ds(s * ROWS_PER_TILE, ROWS_PER_TILE)])
        plsc.subcore_barrier()
        base0 = s * _EDGES_PER_TILE_SEG

        def idx_start(k, b):
            pltpu.async_copy(src_hbm.at[pl.ds(base0 + k * EB, EB)], srcv[b], isem[b])
            pltpu.async_copy(dst_hbm.at[pl.ds(base0 + k * EB, EB)], dstv[b], isem[b])

        idx_start(0, 0)

        @pl.loop(0, T // 2)
        def _(ko):
            for b in range(2):
                nb = 1 - b
                k = ko * 2 + b
                pltpu.make_async_copy(src_hbm.at[pl.ds(base0, EB)], srcv[b], isem[b]).wait()
                pltpu.make_async_copy(dst_hbm.at[pl.ds(base0, EB)], dstv[b], isem[b]).wait()
                gd = pltpu.async_copy(m4c.at[srcv[b]], rows[b], gsem[b])

                @pl.when(k > 0)
                def _():
                    pltpu.make_async_copy(rows[nb], acc.at[dstv[nb]],
                                          ssem[nb]).wait()

                @pl.when(k + 1 < T)
                def _():
                    idx_start(k + 1, nb)

                gd.wait()
                pltpu.async_copy(rows[b], acc.at[dstv[b]], ssem[b], add=True)

        pltpu.make_async_copy(rows[1], acc.at[dstv[1]], ssem[1]).wait()
        tb = base0 + T * EB
        pltpu.sync_copy(src_hbm.at[pl.ds(tb, EB_T)], srcv_t)
        pltpu.sync_copy(dst_hbm.at[pl.ds(tb, EB_T)], dstv_t)
        pltpu.async_copy(m4c.at[srcv_t], rows_t, gsem[0]).wait()
        pltpu.sync_copy(rows_t, acc.at[dstv_t], add=True)
        plsc.subcore_barrier()
        pltpu.sync_copy(acc.at[pl.ds(s * ROWS_PER_TILE, ROWS_PER_TILE)],
                        s4_hbm.at[chunk, pl.ds(s * ROWS_PER_TILE, ROWS_PER_TILE)])
        plsc.subcore_barrier()

    for chunk in range(NCH):
        @pl.when(c == chunk // 2)
        def _(chunk=chunk):
            do_chunk(chunk)


def _segment_sum(m4, src, dst, zer):
    k = pl.kernel(
        _seg_kernel_body,
        out_type=jax.ShapeDtypeStruct((NCH, N_PAD, CW), jnp.float32),
        mesh=_sc_mesh(),
        scratch_types=[
            pltpu.VMEM((EB,), jnp.int32),
            pltpu.VMEM((EB,), jnp.int32),
            pltpu.VMEM((EB,), jnp.int32),
            pltpu.VMEM((EB,), jnp.int32),
            pltpu.VMEM((EB, CW), jnp.float32),
            pltpu.VMEM((EB, CW), jnp.float32),
            pltpu.VMEM((EB_T,), jnp.int32),
            pltpu.VMEM((EB_T,), jnp.int32),
            pltpu.VMEM((EB_T, CW), jnp.float32),
            pltpu.VMEM_SHARED((N_PAD, CW), jnp.float32),
            pltpu.SemaphoreType.DMA,
            pltpu.SemaphoreType.DMA,
            pltpu.SemaphoreType.DMA,
            pltpu.SemaphoreType.DMA,
            pltpu.SemaphoreType.DMA,
            pltpu.SemaphoreType.DMA,
        ],
    )
    return k(m4, src, dst, zer)


def kernel(x, edge_index, batch, lin0_w, lin0_b, p, msn_scale, mlp_w, mlp_b,
           bn_g, bn_b, fc1_w, fc1_b, bn4_g, bn4_b, fc2_w, fc2_b):
    src = edge_index[0]
    dst = edge_index[1]
    ones_e = jnp.ones((EB_C, CW), jnp.float32)
    zer = jnp.zeros((ROWS_PER_TILE, CW), jnp.float32)

    cnt2 = _degree_counts(dst, ones_e, zer)

    h, m4 = _lin0_msg(x, lin0_w, lin0_b[None], _scal(p[0]))
    jk = None
    for i in range(N_LAYERS):
        s4 = _segment_sum(m4, src, dst, zer)
        sc_i = _scal(p[i], 1.0 / p[i], msn_scale[i])
        z, st = _conv_mlp(h, s4, cnt2, mlp_w[i].reshape(NCH, CW, D),
                          mlp_b[i][None], sc_i)
        want_m = i < N_LAYERS - 1
        sc_n = _scal(p[i + 1] if want_m else 0.0)
        outs = _bn_relu(z, st, bn_g[i][None], bn_b[i][None], sc_n, jk, want_m)
        if want_m:
            jk, h, m4 = outs
        else:
            jk = outs[0]

    w1s = fc1_w.reshape(2, D, D)
    out = _pool_head(jk, batch[:, None], w1s, fc1_b[None], bn4_g[None],
                     bn4_b[None], fc2_w, fc2_b[None])
    return out
